# baseline (device time: 90237 ns/iter reference)
import jax
import jax.numpy as jnp
from jax import lax
from jax.experimental import pallas as pl
from jax.experimental.pallas import tpu as pltpu

N_DEV = 16
B, S, C_OUT = 4, 1024, 512
ROWS = B * S
G, U, R = 4, 8, 128

PRF, PRB, ZRF, ZRB, ZAF, ZAB, PAF, PAB = (i * 3 for i in range(8))
NSEM = 24


def kernel(x, k, Wp):
    c_loc = x.shape[2]
    f32 = jnp.float32
    bf16 = jnp.bfloat16

    def body(x_ref, k_ref, w_ref, out_ref, acc_ref, ag_ref,
             pcomm_f, pcomm_b, pstage_f, pstage_b,
             zcomm_f, zcomm_b, zstage_f, zstage_b,
             send_sems, recv_sems):
        me = lax.axis_index("i")
        p = me // 4
        q = lax.rem(me, 4)
        plane_r = p * 4 + lax.rem(q + 1, 4)
        plane_l = p * 4 + lax.rem(q + 3, 4)
        z_r = lax.rem(p + 1, 4) * 4 + q
        z_l = lax.rem(p + 3, 4) * 4 + q

        barrier = pltpu.get_barrier_semaphore()
        for nbr in (plane_l, plane_r, z_l, z_r):
            pl.semaphore_signal(barrier, inc=1, device_id=(nbr,),
                                device_id_type=pl.DeviceIdType.MESH)
        pl.semaphore_wait(barrier, 4)

        kv = k_ref[:, :].astype(bf16)
        wb = w_ref[:, :].astype(bf16)

        def compute_macro(b):
            xb = x_ref[b].astype(bf16)
            zpad = jnp.zeros((3, c_loc), bf16)
            xp = jnp.concatenate([zpad, xb], axis=0)
            conv = (xp[3:, :] * kv[3][None, :]
                    + xp[2:-1, :] * kv[2][None, :]
                    + xp[1:-2, :] * kv[1][None, :]
                    + xp[0:-3, :] * kv[0][None, :])
            a = conv * (jnp.bfloat16(1.0) / (jnp.bfloat16(1.0) + jnp.exp(-conv)))
            part = jnp.dot(a, wb, preferred_element_type=f32)
            acc_ref[pl.ds(8 * b, 8)] = part.reshape(U, R, C_OUT)

        def rdma(src, dst, slot, dev):
            return pltpu.make_async_remote_copy(
                src_ref=src, dst_ref=dst,
                send_sem=send_sems.at[slot], recv_sem=recv_sems.at[slot],
                device_id=(dev,), device_id_type=pl.DeviceIdType.MESH,
            )

        def prs_start(s):
            gf_s = lax.rem(q - s + 4, 4)
            gb_s = lax.rem(q + s, 4)
            pstage_f[s] = acc_ref[pl.ds(8 * gf_s, 4)].astype(bf16)
            pstage_b[s] = acc_ref[pl.ds(8 * gb_s + 4, 4)].astype(bf16)
            cf = rdma(pstage_f.at[s], pcomm_f.at[s], PRF + s, plane_r)
            cb = rdma(pstage_b.at[s], pcomm_b.at[s], PRB + s, plane_l)
            cf.start()
            cb.start()
            return cf, cb

        def prs_finish(s, cf, cb):
            gf_r = lax.rem(q - s + 3, 4)
            gb_r = lax.rem(q + s + 1, 4)
            cf.wait()
            cb.wait()
            fr = pl.ds(8 * gf_r, 4)
            br = pl.ds(8 * gb_r + 4, 4)
            acc_ref[fr] = acc_ref[fr] + pcomm_f[s].astype(f32)
            acc_ref[br] = acc_ref[br] + pcomm_b[s].astype(f32)

        compute_macro(q)
        cf0, cb0 = prs_start(0)
        compute_macro(lax.rem(q + 3, 4))
        compute_macro(lax.rem(q + 1, 4))
        prs_finish(0, cf0, cb0)
        cf1, cb1 = prs_start(1)
        compute_macro(lax.rem(q + 2, 4))
        prs_finish(1, cf1, cb1)
        cf2, cb2 = prs_start(2)
        prs_finish(2, cf2, cb2)

        g_a = lax.rem(q + 1, 4)
        g_b = lax.rem(q + 3, 4)

        for s in range(3):
            jf_s = lax.rem(p - s + 4, 4)
            jf_r = lax.rem(p - s + 3, 4)
            jb_s = lax.rem(p + s, 4)
            jb_r = lax.rem(p + s + 1, 4)
            zstage_f[s] = acc_ref[8 * g_a + jf_s].astype(bf16)
            zstage_b[s] = acc_ref[8 * g_b + 4 + jb_s].astype(bf16)
            cf = rdma(zstage_f.at[s], zcomm_f.at[s], ZRF + s, z_r)
            cb = rdma(zstage_b.at[s], zcomm_b.at[s], ZRB + s, z_l)
            cf.start()
            cb.start()
            cf.wait()
            cb.wait()
            mf = 8 * g_a + jf_r
            mb = 8 * g_b + 4 + jb_r
            acc_ref[mf] = acc_ref[mf] + zcomm_f[s].astype(f32)
            acc_ref[mb] = acc_ref[mb] + zcomm_b[s].astype(f32)

        j_a = lax.rem(p + 1, 4)
        j_b = lax.rem(p + 3, 4)
        ag_ref[8 * g_a + j_a] = acc_ref[8 * g_a + j_a].astype(bf16)
        ag_ref[8 * g_b + 4 + j_b] = acc_ref[8 * g_b + 4 + j_b].astype(bf16)

        for s in range(3):
            mf = 8 * g_a + lax.rem(p + 1 - s + 4, 4)
            mb = 8 * g_b + 4 + lax.rem(p + 3 + s, 4)
            cf = rdma(ag_ref.at[mf], ag_ref.at[mf], ZAF + s, z_r)
            cb = rdma(ag_ref.at[mb], ag_ref.at[mb], ZAB + s, z_l)
            cf.start()
            cb.start()
            cf.wait()
            cb.wait()

        for s in range(3):
            df = pl.ds(8 * lax.rem(q + 1 - s + 4, 4), 4)
            db = pl.ds(8 * lax.rem(q + 3 + s, 4) + 4, 4)
            cf = rdma(ag_ref.at[df], ag_ref.at[df], PAF + s, plane_r)
            cb = rdma(ag_ref.at[db], ag_ref.at[db], PAB + s, plane_l)
            cf.start()
            cb.start()
            cf.wait()
            cb.wait()

        out_ref[...] = ag_ref[...].astype(f32).reshape(B, S, C_OUT)

    return pl.pallas_call(
        body,
        out_shape=jax.ShapeDtypeStruct((B, S, C_OUT), jnp.float32),
        in_specs=[pl.BlockSpec(memory_space=pltpu.VMEM)] * 3,
        out_specs=pl.BlockSpec(memory_space=pltpu.VMEM),
        scratch_shapes=[
            pltpu.VMEM((G * U, R, C_OUT), f32),
            pltpu.VMEM((G * U, R, C_OUT), bf16),
            pltpu.VMEM((3, 4, R, C_OUT), bf16),
            pltpu.VMEM((3, 4, R, C_OUT), bf16),
            pltpu.VMEM((3, 4, R, C_OUT), bf16),
            pltpu.VMEM((3, 4, R, C_OUT), bf16),
            pltpu.VMEM((3, R, C_OUT), bf16),
            pltpu.VMEM((3, R, C_OUT), bf16),
            pltpu.VMEM((3, R, C_OUT), bf16),
            pltpu.VMEM((3, R, C_OUT), bf16),
            pltpu.SemaphoreType.DMA((NSEM,)),
            pltpu.SemaphoreType.DMA((NSEM,)),
        ],
        compiler_params=pltpu.CompilerParams(collective_id=0),
    )(x, k, Wp)


# device time: 89605 ns/iter; 1.0071x vs baseline; 1.0071x over previous
import jax
import jax.numpy as jnp
from jax import lax
from jax.experimental import pallas as pl
from jax.experimental.pallas import tpu as pltpu

N_DEV = 16
B, S, C_OUT = 4, 1024, 512
ROWS = B * S
G, U, R = 4, 8, 128
OFF = (0, 2, 4, 6)

NSEM = 48


def kernel(x, k, Wp):
    c_loc = x.shape[2]
    f32 = jnp.float32
    bf16 = jnp.bfloat16

    def body(x_ref, k_ref, w_ref, out_ref, acc_ref, ag_ref,
             stage1, comm1, stage2, comm2, send_sems, recv_sems):
        me = lax.axis_index("i")
        p = me // 4
        q = lax.rem(me, 4)
        plane_r = p * 4 + lax.rem(q + 1, 4)
        plane_l = p * 4 + lax.rem(q + 3, 4)
        z_r = lax.rem(p + 1, 4) * 4 + q
        z_l = lax.rem(p + 3, 4) * 4 + q
        DEVS = (plane_r, plane_l, z_r, z_l)

        barrier = pltpu.get_barrier_semaphore()
        for nbr in DEVS:
            pl.semaphore_signal(barrier, inc=1, device_id=(nbr,),
                                device_id_type=pl.DeviceIdType.MESH)
        pl.semaphore_wait(barrier, 4)

        kv = k_ref[:, :]
        wb = w_ref[:, :].astype(bf16)

        def compute_macro(b):
            xb = x_ref[b]
            zpad = jnp.zeros((3, c_loc), xb.dtype)
            xp = jnp.concatenate([zpad, xb], axis=0)
            conv = (xp[3:, :] * kv[3][None, :]
                    + xp[2:-1, :] * kv[2][None, :]
                    + xp[1:-2, :] * kv[1][None, :]
                    + xp[0:-3, :] * kv[0][None, :])
            a = conv * (1.0 / (1.0 + jnp.exp(-conv)))
            part = jnp.dot(a.astype(bf16), wb, preferred_element_type=f32)
            acc_ref[pl.ds(8 * b, 8)] = part.reshape(U, 2, 64, C_OUT)

        def rdma(src, dst, slot, dev):
            return pltpu.make_async_remote_copy(
                src_ref=src, dst_ref=dst,
                send_sem=send_sems.at[slot], recv_sem=recv_sems.at[slot],
                device_id=(dev,), device_id_type=pl.DeviceIdType.MESH,
            )

        def blk(ref, g, o):
            return pl.ds(8 * g + o, 2)

        def s1_stage_start(qi, s, gs, dev):
            stage1[qi, s] = acc_ref[blk(None, gs, OFF[qi])].astype(bf16)
            c = rdma(stage1.at[qi, s], comm1.at[qi, s], qi * 3 + s, dev)
            c.start()
            return c

        def s1_finish(qi, s, gr, c):
            c.wait()
            d = blk(None, gr, OFF[qi])
            acc_ref[d] = acc_ref[d] + comm1[qi, s].astype(f32)

        def fsend(i, s):
            return lax.rem(i - s + 4, 4)

        def frecv(i, s):
            return lax.rem(i - s + 3, 4)

        def bsend(i, s):
            return lax.rem(i + s, 4)

        def brecv(i, s):
            return lax.rem(i + s + 1, 4)

        compute_macro(q)
        c1 = s1_stage_start(0, 0, q, plane_r)
        c2 = s1_stage_start(1, 0, q, plane_l)
        compute_macro(lax.rem(q + 1, 4))
        compute_macro(lax.rem(q + 2, 4))
        compute_macro(lax.rem(q + 3, 4))
        c3 = s1_stage_start(2, 0, fsend(p, 0), z_r)
        c4 = s1_stage_start(3, 0, bsend(p, 0), z_l)
        cs = (c1, c2, c3, c4)
        for s in range(3):
            if s > 0:
                cs = (s1_stage_start(0, s, fsend(q, s), plane_r),
                      s1_stage_start(1, s, bsend(q, s), plane_l),
                      s1_stage_start(2, s, fsend(p, s), z_r),
                      s1_stage_start(3, s, bsend(p, s), z_l))
            s1_finish(0, s, frecv(q, s), cs[0])
            s1_finish(1, s, brecv(q, s), cs[1])
            s1_finish(2, s, frecv(p, s), cs[2])
            s1_finish(3, s, brecv(p, s), cs[3])

        gA = lax.rem(q + 1, 4)
        gB = lax.rem(q + 3, 4)
        gC = lax.rem(p + 1, 4)
        gD = lax.rem(p + 3, 4)
        GOWN = (gA, gB, gC, gD)
        S2DEV = (z_r, z_l, plane_r, plane_l)
        S2IDX = (p, p, q, q)
        S2FWD = (True, False, True, False)

        def uidx(g, o, j):
            return (8 * g + o + j // 2, lax.rem(j, 2))

        for s in range(3):
            cs2 = []
            for qi in range(4):
                i = S2IDX[qi]
                js = fsend(i, s) if S2FWD[qi] else bsend(i, s)
                m, h = uidx(GOWN[qi], OFF[qi], js)
                stage2[qi, s] = acc_ref[m, h].astype(bf16)
                c = rdma(stage2.at[qi, s], comm2.at[qi, s],
                         12 + qi * 3 + s, S2DEV[qi])
                c.start()
                cs2.append(c)
            for c in cs2:
                c.wait()
            for qi in range(4):
                i = S2IDX[qi]
                jr = frecv(i, s) if S2FWD[qi] else brecv(i, s)
                m, h = uidx(GOWN[qi], OFF[qi], jr)
                acc_ref[m, h] = acc_ref[m, h] + comm2[qi, s].astype(f32)

        JOWN = (lax.rem(p + 1, 4), lax.rem(p + 3, 4),
                lax.rem(q + 1, 4), lax.rem(q + 3, 4))
        for qi in range(4):
            m, h = uidx(GOWN[qi], OFF[qi], JOWN[qi])
            ag_ref[m, h] = acc_ref[m, h].astype(bf16)

        def asend_f(i, s):
            return lax.rem(i + 1 - s + 4, 4)

        def asend_b(i, s):
            return lax.rem(i + 3 + s, 4)

        for s in range(3):
            cs3 = []
            for qi in range(4):
                i = S2IDX[qi]
                j = asend_f(i, s) if S2FWD[qi] else asend_b(i, s)
                m, h = uidx(GOWN[qi], OFF[qi], j)
                u = ag_ref.at[m, h]
                c = rdma(u, u, 24 + qi * 3 + s, S2DEV[qi])
                c.start()
                cs3.append(c)
            for c in cs3:
                c.wait()

        S4DEV = DEVS
        S4IDX = (q, q, p, p)
        for s in range(3):
            cs4 = []
            for qi in range(4):
                i = S4IDX[qi]
                g = asend_f(i, s) if S2FWD[qi] else asend_b(i, s)
                bref = ag_ref.at[blk(None, g, OFF[qi])]
                c = rdma(bref, bref, 36 + qi * 3 + s, S4DEV[qi])
                c.start()
                cs4.append(c)
            for c in cs4:
                c.wait()

        out_ref[...] = ag_ref[...].astype(f32).reshape(B, S, C_OUT)

    return pl.pallas_call(
        body,
        out_shape=jax.ShapeDtypeStruct((B, S, C_OUT), jnp.float32),
        in_specs=[pl.BlockSpec(memory_space=pltpu.VMEM)] * 3,
        out_specs=pl.BlockSpec(memory_space=pltpu.VMEM),
        scratch_shapes=[
            pltpu.VMEM((G * U, 2, 64, C_OUT), f32),
            pltpu.VMEM((G * U, 2, 64, C_OUT), bf16),
            pltpu.VMEM((4, 3, 2, 2, 64, C_OUT), bf16),
            pltpu.VMEM((4, 3, 2, 2, 64, C_OUT), bf16),
            pltpu.VMEM((4, 3, 64, C_OUT), bf16),
            pltpu.VMEM((4, 3, 64, C_OUT), bf16),
            pltpu.SemaphoreType.DMA((NSEM,)),
            pltpu.SemaphoreType.DMA((NSEM,)),
        ],
        compiler_params=pltpu.CompilerParams(collective_id=0),
    )(x, k, Wp)


# device time: 87787 ns/iter; 1.0279x vs baseline; 1.0207x over previous
import jax
import jax.numpy as jnp
from jax import lax
from jax.experimental import pallas as pl
from jax.experimental.pallas import tpu as pltpu

N_DEV = 16
B, S, C_OUT = 4, 1024, 512
ROWS = B * S
G, U, R = 4, 8, 128

PRF, PRB, PAF, PAB = 0, 3, 6, 9
Z1, Z2 = 12, 13
ZA1, ZA2 = 14, 16
NSEM = 20


def kernel(x, k, Wp):
    c_loc = x.shape[2]
    f32 = jnp.float32
    bf16 = jnp.bfloat16

    def body(x_ref, k_ref, w_ref, out_ref, acc_ref, ag_ref,
             pcomm_f, pcomm_b, zst1, zcomm1, zst2, zcomm2,
             send_sems, recv_sems):
        me = lax.axis_index("i")
        p = me // 4
        q = lax.rem(me, 4)
        p0 = lax.rem(p, 2)
        p1 = p // 2
        plane_r = p * 4 + lax.rem(q + 1, 4)
        plane_l = p * 4 + lax.rem(q + 3, 4)
        px1 = p + 1 - 2 * p0
        px2 = lax.rem(p + 2, 4)
        zd1 = px1 * 4 + q
        zd2 = px2 * 4 + q

        barrier = pltpu.get_barrier_semaphore()
        for nbr in (plane_l, plane_r, zd1, zd2):
            pl.semaphore_signal(barrier, inc=1, device_id=(nbr,),
                                device_id_type=pl.DeviceIdType.MESH)
        pl.semaphore_wait(barrier, 4)

        kv = k_ref[:, :]
        wb = w_ref[:, :].astype(bf16)

        def compute_macro(b):
            xb = x_ref[b]
            zpad = jnp.zeros((3, c_loc), xb.dtype)
            xp = jnp.concatenate([zpad, xb], axis=0)
            conv = (xp[3:, :] * kv[3][None, :]
                    + xp[2:-1, :] * kv[2][None, :]
                    + xp[1:-2, :] * kv[1][None, :]
                    + xp[0:-3, :] * kv[0][None, :])
            a = conv * (1.0 / (1.0 + jnp.exp(-conv)))
            part = jnp.dot(a.astype(bf16), wb, preferred_element_type=f32)
            acc_ref[pl.ds(8 * b, 8)] = part.reshape(U, R, C_OUT).astype(bf16)

        def rdma(src, dst, slot, dev):
            return pltpu.make_async_remote_copy(
                src_ref=src, dst_ref=dst,
                send_sem=send_sems.at[slot], recv_sem=recv_sems.at[slot],
                device_id=(dev,), device_id_type=pl.DeviceIdType.MESH,
            )

        def prs_start(s):
            gf_s = lax.rem(q - s + 4, 4)
            gb_s = lax.rem(q + s, 4)
            cf = rdma(acc_ref.at[pl.ds(8 * gf_s, 4)], pcomm_f.at[s],
                      PRF + s, plane_r)
            cb = rdma(acc_ref.at[pl.ds(8 * gb_s + 4, 4)], pcomm_b.at[s],
                      PRB + s, plane_l)
            cf.start()
            cb.start()
            return cf, cb

        def prs_finish(s, cf, cb):
            cf.wait_recv()
            cb.wait_recv()
            fr = pl.ds(8 * lax.rem(q - s + 3, 4), 4)
            br = pl.ds(8 * lax.rem(q + s + 1, 4) + 4, 4)
            acc_ref[fr] = acc_ref[fr] + pcomm_f[s]
            acc_ref[br] = acc_ref[br] + pcomm_b[s]
            cf.wait_send()
            cb.wait_send()

        compute_macro(q)
        cf0, cb0 = prs_start(0)
        compute_macro(lax.rem(q + 3, 4))
        compute_macro(lax.rem(q + 1, 4))
        prs_finish(0, cf0, cb0)
        cf1, cb1 = prs_start(1)
        compute_macro(lax.rem(q + 2, 4))
        prs_finish(1, cf1, cb1)
        cf2, cb2 = prs_start(2)
        prs_finish(2, cf2, cb2)

        gA = lax.rem(q + 1, 4)
        gB = lax.rem(q + 3, 4)
        lo = 8 * gA
        hi = 8 * gB + 4

        zst1[0] = acc_ref[lo + 1 - p0]
        zst1[1] = acc_ref[lo + 3 - p0]
        zst1[2] = acc_ref[hi + 1 - p0]
        zst1[3] = acc_ref[hi + 3 - p0]
        c = rdma(zst1, zcomm1, Z1, zd1)
        c.start()
        c.wait_recv()
        acc_ref[lo + p0] = acc_ref[lo + p0] + zcomm1[0]
        acc_ref[lo + p0 + 2] = acc_ref[lo + p0 + 2] + zcomm1[1]
        acc_ref[hi + p0] = acc_ref[hi + p0] + zcomm1[2]
        acc_ref[hi + p0 + 2] = acc_ref[hi + p0 + 2] + zcomm1[3]
        c.wait_send()
        j2 = p0 + 2 * (1 - p1)
        zst2[0] = acc_ref[lo + j2]
        zst2[1] = acc_ref[hi + j2]
        c = rdma(zst2, zcomm2, Z2, zd2)
        c.start()
        c.wait_recv()
        acc_ref[lo + p] = acc_ref[lo + p] + zcomm2[0]
        acc_ref[hi + p] = acc_ref[hi + p] + zcomm2[1]
        c.wait_send()

        ag_ref[lo + p] = acc_ref[lo + p]
        ag_ref[hi + p] = acc_ref[hi + p]

        c1 = rdma(ag_ref.at[lo + p], ag_ref.at[lo + p], ZA1, zd2)
        c2 = rdma(ag_ref.at[hi + p], ag_ref.at[hi + p], ZA1 + 1, zd2)
        c1.start()
        c2.start()
        c1.wait()
        c2.wait()
        cs = [rdma(ag_ref.at[m], ag_ref.at[m], ZA2 + i, zd1)
              for i, m in enumerate((lo + p, lo + px2, hi + p, hi + px2))]
        for c in cs:
            c.start()
        for c in cs:
            c.wait()

        out_ref[gA, pl.ds(0, 512)] = (
            ag_ref[pl.ds(lo, 4)].astype(f32).reshape(512, C_OUT))
        out_ref[gB, pl.ds(512, 512)] = (
            ag_ref[pl.ds(hi, 4)].astype(f32).reshape(512, C_OUT))

        for s in range(3):
            gf = lax.rem(q + 1 - s + 4, 4)
            gb = lax.rem(q + 3 + s, 4)
            df = pl.ds(8 * gf, 4)
            db = pl.ds(8 * gb + 4, 4)
            cf = rdma(ag_ref.at[df], ag_ref.at[df], PAF + s, plane_r)
            cb = rdma(ag_ref.at[db], ag_ref.at[db], PAB + s, plane_l)
            cf.start()
            cb.start()
            cf.wait()
            cb.wait()
            grf = lax.rem(q - s + 4, 4)
            grb = lax.rem(q + s, 4)
            out_ref[grf, pl.ds(0, 512)] = (
                ag_ref[pl.ds(8 * grf, 4)].astype(f32).reshape(512, C_OUT))
            out_ref[grb, pl.ds(512, 512)] = (
                ag_ref[pl.ds(8 * grb + 4, 4)].astype(f32).reshape(512, C_OUT))

    return pl.pallas_call(
        body,
        out_shape=jax.ShapeDtypeStruct((B, S, C_OUT), jnp.float32),
        in_specs=[pl.BlockSpec(memory_space=pltpu.VMEM)] * 3,
        out_specs=pl.BlockSpec(memory_space=pltpu.VMEM),
        scratch_shapes=[
            pltpu.VMEM((G * U, R, C_OUT), bf16),
            pltpu.VMEM((G * U, R, C_OUT), bf16),
            pltpu.VMEM((3, 4, R, C_OUT), bf16),
            pltpu.VMEM((3, 4, R, C_OUT), bf16),
            pltpu.VMEM((4, R, C_OUT), bf16),
            pltpu.VMEM((4, R, C_OUT), bf16),
            pltpu.VMEM((2, R, C_OUT), bf16),
            pltpu.VMEM((2, R, C_OUT), bf16),
            pltpu.SemaphoreType.DMA((NSEM,)),
            pltpu.SemaphoreType.DMA((NSEM,)),
        ],
        compiler_params=pltpu.CompilerParams(collective_id=0),
    )(x, k, Wp)
